# Initial kernel scaffold; baseline (speedup 1.0000x reference)
import jax
import jax.numpy as jnp
from jax.experimental import pallas as pl


def kernel(teacher_logits, W_emb, W_out, b_out, input_ids, token_type_ids, attention_mask, nn_mask, example_indices, augmented_indices, nn_ranks, augment_rank, temperature):
    raise NotImplementedError("write your pallas kernel here")


# speedup vs baseline: 7.0391x; 7.0391x over previous
import functools

import jax
import jax.numpy as jnp
from jax import lax
from jax.experimental import pallas as pl
from jax.experimental.pallas import tpu as pltpu
from jax.experimental.pallas import tpu_sc as plsc

NC, NS = 2, 16
NW = NC * NS
B, S = 2048, 128
G, GS = 128, 16
NL = 64
BPW = B // NW


def _vmat_body(a_ref, w_ref, o_ref):
    dot = jnp.dot(a_ref[...], w_ref[...], preferred_element_type=jnp.float32)
    o_ref[...] = jnp.concatenate(
        [dot, jnp.zeros(dot.shape, jnp.float32)], axis=1)


def _precompute_v(w_emb, w_out_s):
    m, d = w_emb.shape
    n = w_out_s.shape[1]
    bm = 1024
    return pl.pallas_call(
        _vmat_body,
        grid=(pl.cdiv(m, bm),),
        in_specs=[pl.BlockSpec((bm, d), lambda i: (i, 0)),
                  pl.BlockSpec((d, n), lambda i: (0, 0))],
        out_specs=pl.BlockSpec((bm, 2 * n), lambda i: (i, 0)),
        out_shape=jax.ShapeDtypeStruct((m, 2 * n), jnp.float32),
    )(w_emb, w_out_s)


_sc_mesh = plsc.VectorSubcoreMesh(core_axis_name="c", subcore_axis_name="s")


@functools.partial(
    pl.kernel,
    mesh=_sc_mesh,
    out_type=(jax.ShapeDtypeStruct((B, NL), jnp.float32),
              jax.ShapeDtypeStruct((B, 2 * NL), jnp.float32)),
    scratch_types=[
        pltpu.VMEM((BPW, S), jnp.int32),
        pltpu.VMEM((BPW,), jnp.int32),
        pltpu.VMEM((BPW, 2 * NL), jnp.float32),
        pltpu.VMEM((2, S, 2 * NL), jnp.float32),
        pltpu.VMEM((BPW, NL), jnp.float32),
        pltpu.SemaphoreType.DMA,
        pltpu.SemaphoreType.DMA,
        pltpu.SemaphoreType.DMA,
    ],
)
def _sc_gather(v_hbm, ids_hbm, tidx_hbm, teatab_hbm, slog_hbm, tea_hbm,
               ids_v, tidx_v, tea_v, buf_v, out_v, sem0, sem1, sem_t):
    wid = lax.axis_index("s") * NC + lax.axis_index("c")
    base = wid * BPW
    pltpu.sync_copy(ids_hbm.at[pl.ds(base, BPW)], ids_v)
    pltpu.sync_copy(tidx_hbm.at[pl.ds(base, BPW)], tidx_v)
    tea_cp = pltpu.async_copy(teatab_hbm.at[tidx_v], tea_v, sem_t)

    pltpu.async_copy(v_hbm.at[ids_v.at[0]], buf_v.at[0], sem0)
    pltpu.async_copy(v_hbm.at[ids_v.at[1]], buf_v.at[1], sem1)

    def _accum(b, e):
        def row_body(r, acc):
            a0, a1, a2, a3 = acc
            a0 = a0 + buf_v[b, r, pl.ds(0, 16)]
            a1 = a1 + buf_v[b, r, pl.ds(16, 16)]
            a2 = a2 + buf_v[b, r, pl.ds(32, 16)]
            a3 = a3 + buf_v[b, r, pl.ds(48, 16)]
            return (a0, a1, a2, a3)

        z = jnp.zeros((16,), jnp.float32)
        a0, a1, a2, a3 = lax.fori_loop(0, S, row_body, (z, z, z, z))
        out_v[e, pl.ds(0, 16)] = a0
        out_v[e, pl.ds(16, 16)] = a1
        out_v[e, pl.ds(32, 16)] = a2
        out_v[e, pl.ds(48, 16)] = a3

    def outer(eo, carry):
        for b, sem in ((0, sem0), (1, sem1)):
            e = eo * 2 + b
            pltpu.make_async_copy(v_hbm.at[ids_v.at[e]], buf_v.at[b],
                                  sem).wait()
            _accum(b, e)

            @pl.when(e + 2 < BPW)
            def _():
                pltpu.async_copy(v_hbm.at[ids_v.at[e + 2]], buf_v.at[b], sem)
        return carry

    lax.fori_loop(0, BPW // 2, outer, 0)
    tea_cp.wait()
    pltpu.sync_copy(out_v, slog_hbm.at[pl.ds(base, BPW)])
    pltpu.sync_copy(tea_v, tea_hbm.at[pl.ds(base, BPW)])


def _select_body(sl_ref, tea_ref, b_ref, ranks_ref, par_ref, tinv_ref,
                 st_ref, sel_ref, sr_ref):
    tinv = tinv_ref[0, 0]
    lp_t = sl_ref[...] + b_ref[...]
    m = jnp.max(lp_t, axis=2, keepdims=True)
    lse = jnp.log(jnp.sum(jnp.exp(lp_t - m), axis=2, keepdims=True)) + m
    log_p = lp_t - lse

    par = par_ref[...] == 1
    tea = jnp.where(par, tea_ref[..., NL:], tea_ref[..., :NL])
    tz = tea * tinv
    mt = jnp.max(tz, axis=2, keepdims=True)
    et = jnp.exp(tz - mt)
    ssum = jnp.sum(et, axis=2, keepdims=True)
    q = et / ssum
    logq = tz - (jnp.log(ssum) + mt)

    d = jnp.sum(q * (logq - log_p), axis=2, keepdims=True)

    idx = lax.broadcasted_iota(jnp.int32, (G, GS, 1), 1)
    m1 = jnp.max(d, axis=1, keepdims=True)
    i1 = jnp.min(jnp.where(d == m1, idx, GS), axis=1, keepdims=True)
    i1 = jnp.minimum(i1, GS - 1)
    d2 = jnp.where(idx == i1, -jnp.inf, d)
    m2 = jnp.max(d2, axis=1, keepdims=True)
    i2 = jnp.min(jnp.where(d2 == m2, idx, GS), axis=1, keepdims=True)
    i2 = jnp.minimum(i2, GS - 1)

    sel_mask = (idx == i2)
    st_ref[...] = jnp.sum(tea * sel_mask.astype(jnp.float32), axis=1)
    gid = lax.broadcasted_iota(jnp.int32, (G, 1, 1), 0)
    sel_ref[...] = gid * GS + i2
    sr_ref[...] = jnp.sum(jnp.where(sel_mask, ranks_ref[...], 0), axis=1,
                          keepdims=True)


def _select(slog3, tea3, b3, ranks2, par2, tinv):
    return pl.pallas_call(
        _select_body,
        in_specs=[pl.BlockSpec(), pl.BlockSpec(), pl.BlockSpec(),
                  pl.BlockSpec(), pl.BlockSpec(),
                  pl.BlockSpec(memory_space=pltpu.SMEM)],
        out_shape=(jax.ShapeDtypeStruct((G, NL), jnp.float32),
                   jax.ShapeDtypeStruct((G, 1, 1), jnp.int32),
                   jax.ShapeDtypeStruct((G, 1, 1), jnp.int32)),
    )(slog3, tea3, b3, ranks2, par2, tinv)


def kernel(teacher_logits, W_emb, W_out, b_out, input_ids, token_type_ids,
           attention_mask, nn_mask, example_indices, augmented_indices,
           nn_ranks, augment_rank, temperature):
    n_ex, n_aug, nl = teacher_logits.shape
    inv_t = (1.0 / temperature).astype(jnp.float32)
    w_out_s = W_out.astype(jnp.float32) * (inv_t / S)
    v = _precompute_v(W_emb.astype(jnp.float32), w_out_s)

    teatab = teacher_logits.reshape(n_ex * n_aug // 2, 2 * nl)
    tflat = (jnp.take(example_indices, nn_mask) * n_aug
             + augmented_indices).astype(jnp.int32)
    slog, tea_pairs = _sc_gather(v, input_ids.astype(jnp.int32),
                                 tflat // 2, teatab)

    b3 = (b_out.astype(jnp.float32) * inv_t).reshape(1, 1, nl)
    st, sel, sr = _select(slog.reshape(G, GS, nl),
                          tea_pairs.reshape(G, GS, 2 * nl),
                          b3, nn_ranks.reshape(G, GS, 1).astype(jnp.int32),
                          (tflat % 2).reshape(G, GS, 1), inv_t.reshape(1, 1))
    return st, sel.reshape(G), sr.reshape(G)


# speedup vs baseline: 8.8915x; 1.2632x over previous
import functools

import jax
import jax.numpy as jnp
from jax import lax
from jax.experimental import pallas as pl
from jax.experimental.pallas import tpu as pltpu
from jax.experimental.pallas import tpu_sc as plsc

NC, NS = 2, 16
NW = NC * NS
B, S = 2048, 128
G, GS = 128, 16
NL = 64
NA = 8
BPW = B // NW
VBLK = 512


def _vmat_body(a_ref, w_ref, o_ref):
    dot = jnp.dot(a_ref[...], w_ref[...], preferred_element_type=jnp.float32)
    o_ref[...] = jnp.concatenate(
        [dot, jnp.zeros(dot.shape, jnp.float32)], axis=1)


def _precompute_v(w_emb, w_out_s):
    m, d = w_emb.shape
    n = w_out_s.shape[1]
    bm = 1024
    return pl.pallas_call(
        _vmat_body,
        grid=(pl.cdiv(m, bm),),
        in_specs=[pl.BlockSpec((bm, d), lambda i: (i, 0)),
                  pl.BlockSpec((d, n), lambda i: (0, 0))],
        out_specs=pl.BlockSpec((bm, 2 * n), lambda i: (i, 0)),
        out_shape=jax.ShapeDtypeStruct((m, 2 * n), jnp.float32),
    )(w_emb, w_out_s)


def _slab_body(n_ex, ex_ref, tea_ref, o_ref):
    k = pl.program_id(0)

    @pl.when(k == 0)
    def _():
        o_ref[...] = jnp.zeros_like(o_ref)

    voc = lax.broadcasted_iota(jnp.int32, (G, VBLK), 1) + k * VBLK
    oh = (ex_ref[...] == voc).astype(jnp.float32)
    rvalid = (lax.broadcasted_iota(jnp.int32, (VBLK, 1, 1), 0)
              + k * VBLK) < n_ex
    tea = jnp.where(rvalid, tea_ref[...], 0.0)
    for a in range(NA):
        o_ref[:, a, :] += jnp.dot(oh, tea[:, a, :],
                                  preferred_element_type=jnp.float32)


def _extract_slabs(ex2, teacher_logits):
    n_ex = teacher_logits.shape[0]
    return pl.pallas_call(
        functools.partial(_slab_body, n_ex),
        grid=(pl.cdiv(n_ex, VBLK),),
        in_specs=[pl.BlockSpec((G, 1), lambda k: (0, 0)),
                  pl.BlockSpec((VBLK, NA, NL), lambda k: (k, 0, 0))],
        out_specs=pl.BlockSpec((G, NA, NL), lambda k: (0, 0, 0)),
        out_shape=jax.ShapeDtypeStruct((G, NA, NL), jnp.float32),
    )(ex2, teacher_logits)


_sc_mesh = plsc.VectorSubcoreMesh(core_axis_name="c", subcore_axis_name="s")


@functools.partial(
    pl.kernel,
    mesh=_sc_mesh,
    out_type=jax.ShapeDtypeStruct((B, NL), jnp.float32),
    scratch_types=[
        pltpu.VMEM((BPW, S), jnp.int32),
        pltpu.VMEM((2, S, 2 * NL), jnp.float32),
        pltpu.VMEM((BPW, NL), jnp.float32),
        pltpu.SemaphoreType.DMA,
        pltpu.SemaphoreType.DMA,
    ],
)
def _sc_gather(v_hbm, ids_hbm, slog_hbm, ids_v, buf_v, out_v, sem0, sem1):
    wid = lax.axis_index("s") * NC + lax.axis_index("c")
    base = wid * BPW
    pltpu.sync_copy(ids_hbm.at[pl.ds(base, BPW)], ids_v)

    pltpu.async_copy(v_hbm.at[ids_v.at[0]], buf_v.at[0], sem0)
    pltpu.async_copy(v_hbm.at[ids_v.at[1]], buf_v.at[1], sem1)

    def _accum(b, e):
        def row_body(i, acc):
            a0, a1, a2, a3 = acc
            r = i * 4
            for u in range(4):
                a0 = a0 + buf_v[b, r + u, pl.ds(0, 16)]
                a1 = a1 + buf_v[b, r + u, pl.ds(16, 16)]
                a2 = a2 + buf_v[b, r + u, pl.ds(32, 16)]
                a3 = a3 + buf_v[b, r + u, pl.ds(48, 16)]
            return (a0, a1, a2, a3)

        z = jnp.zeros((16,), jnp.float32)
        a0, a1, a2, a3 = lax.fori_loop(0, S // 4, row_body, (z, z, z, z))
        out_v[e, pl.ds(0, 16)] = a0
        out_v[e, pl.ds(16, 16)] = a1
        out_v[e, pl.ds(32, 16)] = a2
        out_v[e, pl.ds(48, 16)] = a3

    def outer(eo, carry):
        for b, sem in ((0, sem0), (1, sem1)):
            e = eo * 2 + b
            pltpu.make_async_copy(v_hbm.at[ids_v.at[e]], buf_v.at[b],
                                  sem).wait()
            _accum(b, e)

            @pl.when(e + 2 < BPW)
            def _():
                pltpu.async_copy(v_hbm.at[ids_v.at[e + 2]], buf_v.at[b], sem)
        return carry

    lax.fori_loop(0, BPW // 2, outer, 0)
    pltpu.sync_copy(out_v, slog_hbm.at[pl.ds(base, BPW)])


def _select_body(sl_ref, slab_ref, b_ref, ranks_ref, aug_ref, tinv_ref,
                 st_ref, sel_ref, sr_ref):
    tinv = tinv_ref[0, 0]
    lp_t = sl_ref[...] + b_ref[...]
    m = jnp.max(lp_t, axis=2, keepdims=True)
    lse = jnp.log(jnp.sum(jnp.exp(lp_t - m), axis=2, keepdims=True)) + m
    log_p = lp_t - lse

    aug = aug_ref[...]
    tea = jnp.zeros((G, GS, NL), jnp.float32)
    for a in range(NA):
        tea = tea + jnp.where(aug == a, 1.0, 0.0) * slab_ref[:, a:a + 1, :]

    tz = tea * tinv
    mt = jnp.max(tz, axis=2, keepdims=True)
    et = jnp.exp(tz - mt)
    ssum = jnp.sum(et, axis=2, keepdims=True)
    q = et / ssum
    logq = tz - (jnp.log(ssum) + mt)

    d = jnp.sum(q * (logq - log_p), axis=2, keepdims=True)

    idx = lax.broadcasted_iota(jnp.int32, (G, GS, 1), 1)
    m1 = jnp.max(d, axis=1, keepdims=True)
    i1 = jnp.min(jnp.where(d == m1, idx, GS), axis=1, keepdims=True)
    i1 = jnp.minimum(i1, GS - 1)
    d2 = jnp.where(idx == i1, -jnp.inf, d)
    m2 = jnp.max(d2, axis=1, keepdims=True)
    i2 = jnp.min(jnp.where(d2 == m2, idx, GS), axis=1, keepdims=True)
    i2 = jnp.minimum(i2, GS - 1)

    sel_mask = (idx == i2)
    st_ref[...] = jnp.sum(tea * sel_mask.astype(jnp.float32), axis=1)
    gid = lax.broadcasted_iota(jnp.int32, (G, 1, 1), 0)
    sel_ref[...] = gid * GS + i2
    sr_ref[...] = jnp.sum(jnp.where(sel_mask, ranks_ref[...], 0), axis=1,
                          keepdims=True)


def _select(slog3, slab3, b3, ranks3, aug3, tinv):
    return pl.pallas_call(
        _select_body,
        in_specs=[pl.BlockSpec(), pl.BlockSpec(), pl.BlockSpec(),
                  pl.BlockSpec(), pl.BlockSpec(),
                  pl.BlockSpec(memory_space=pltpu.SMEM)],
        out_shape=(jax.ShapeDtypeStruct((G, NL), jnp.float32),
                   jax.ShapeDtypeStruct((G, 1, 1), jnp.int32),
                   jax.ShapeDtypeStruct((G, 1, 1), jnp.int32)),
    )(slog3, slab3, b3, ranks3, aug3, tinv)


def kernel(teacher_logits, W_emb, W_out, b_out, input_ids, token_type_ids,
           attention_mask, nn_mask, example_indices, augmented_indices,
           nn_ranks, augment_rank, temperature):
    nl = teacher_logits.shape[2]
    inv_t = (1.0 / temperature).astype(jnp.float32)
    w_out_s = W_out.astype(jnp.float32) * (inv_t / S)
    v = _precompute_v(W_emb.astype(jnp.float32), w_out_s)

    slabs = _extract_slabs(example_indices.astype(jnp.int32).reshape(G, 1),
                           teacher_logits)
    slog = _sc_gather(v, input_ids.astype(jnp.int32))

    b3 = (b_out.astype(jnp.float32) * inv_t).reshape(1, 1, nl)
    st, sel, sr = _select(slog.reshape(G, GS, nl), slabs, b3,
                          nn_ranks.reshape(G, GS, 1).astype(jnp.int32),
                          augmented_indices.astype(jnp.int32).reshape(G, GS, 1),
                          inv_t.reshape(1, 1))
    return st, sel.reshape(G), sr.reshape(G)
